# baseline (speedup 1.0000x reference)
import jax
import jax.numpy as jnp
from jax.experimental import pallas as pl
from jax.experimental.pallas import tpu as pltpu

DIM = 1024
NS = 64
PROJ_ROWS = 512
B_BLK = 8
LW = B_BLK * NS
TC = 128


def _proj_kernel(x_ref, w_in_ref, w_cat_ref, bias_ref,
                 k_ref, q_ref, v_ref, ax_ref):
    xb = x_ref[...].reshape(2 * PROJ_ROWS, DIM)
    xp = jnp.dot(xb, w_in_ref[...], preferred_element_type=jnp.float32)
    xp = xp * jax.nn.sigmoid(xp)
    kvqa = jnp.dot(xp.astype(jnp.bfloat16), w_cat_ref[...],
                   preferred_element_type=jnp.float32)
    kvqa = kvqa + bias_ref[...]
    k_ref[...] = kvqa[:, 0:NS].reshape(2, PROJ_ROWS, NS)
    q_ref[...] = kvqa[:, 2 * NS:3 * NS].reshape(2, PROJ_ROWS, NS)
    v_ref[...] = jnp.concatenate(
        [kvqa[0:PROJ_ROWS, NS:2 * NS],
         kvqa[PROJ_ROWS:2 * PROJ_ROWS, NS:2 * NS]], axis=1)
    ax_ref[...] = jnp.concatenate(
        [kvqa[0:PROJ_ROWS, 3 * NS:4 * NS],
         kvqa[PROJ_ROWS:2 * PROJ_ROWS, 3 * NS:4 * NS]], axis=1)


def _scan_kernel(kt_ref, qt_ref, v_ref, ax_ref, d_ref, e8_ref, wout_ref,
                 out_ref, sfin_ref, s_ref, o_ref):
    @pl.when(pl.program_id(1) == 0)
    def _():
        s_ref[...] = jnp.zeros_like(s_ref)

    d_row = d_ref[...]
    e8 = e8_ref[...]
    cdims = (((0,), (0,)), ((), ()))

    def body(tt, S):
        base = tt * B_BLK
        kt_t = kt_ref[0, pl.ds(base, B_BLK), :]
        kbt = jax.lax.dot_general(kt_t, e8, cdims,
                                  preferred_element_type=jnp.float32)
        r = jnp.sum(S * kbt, axis=0, keepdims=True)
        z = ax_ref[pl.ds(tt, 1), :] + d_row * r
        alpha = jax.nn.sigmoid(z)
        w = (1.0 - alpha) * v_ref[pl.ds(tt, 1), :]
        S = alpha * S + w * kbt
        qt_t = qt_ref[0, pl.ds(base, B_BLK), :]
        qbt = jax.lax.dot_general(qt_t, e8, cdims,
                                  preferred_element_type=jnp.float32)
        o = jnp.sum(S * qbt, axis=0, keepdims=True)
        o = o * o * jax.nn.sigmoid(o)
        o_ref[pl.ds(tt, 1), :] = o
        return S

    S = jax.lax.fori_loop(0, TC, body, s_ref[...], unroll=128)
    s_ref[...] = S

    @pl.when(pl.program_id(1) == pl.num_programs(1) - 1)
    def _():
        for b in range(B_BLK):
            sfin_ref[b, :, :] = S[:, b * NS:(b + 1) * NS].T

    cell = jnp.concatenate(
        [o_ref[:, b * NS:(b + 1) * NS] for b in range(B_BLK)], axis=0)
    out = jnp.dot(cell, wout_ref[...], preferred_element_type=jnp.float32)
    out_ref[...] = out.reshape(B_BLK, TC, DIM)


def kernel(x, W_in, W_k, W_v, W_q, W_alpha, d_alpha, b_alpha, W_out):
    B, T, D = x.shape
    W_in_T = W_in.T
    W_cat = jnp.concatenate([W_k, W_v, W_q, W_alpha], axis=0).T
    bias = jnp.concatenate(
        [jnp.zeros((3 * NS,), jnp.float32), b_alpha])[None, :]

    tpb = T // PROJ_ROWS
    grid_a = ((B // 2) * tpb,)
    nr_spec = pl.BlockSpec((2, PROJ_ROWS, NS),
                           lambda i: (i // tpb, i % tpb, 0))
    rows_spec = pl.BlockSpec((PROJ_ROWS, 2 * NS), lambda i: (i % tpb, i // tpb))
    k2, q2, v_rows, ax_rows = pl.pallas_call(
        _proj_kernel,
        grid=grid_a,
        in_specs=[
            pl.BlockSpec((2, PROJ_ROWS, DIM), lambda i: (i // tpb, i % tpb, 0)),
            pl.BlockSpec((DIM, DIM), lambda i: (0, 0)),
            pl.BlockSpec((DIM, 4 * NS), lambda i: (0, 0)),
            pl.BlockSpec((1, 4 * NS), lambda i: (0, 0)),
        ],
        out_specs=[nr_spec, nr_spec, rows_spec, rows_spec],
        out_shape=[
            jax.ShapeDtypeStruct((B, T, NS), jnp.float32),
            jax.ShapeDtypeStruct((B, T, NS), jnp.float32),
            jax.ShapeDtypeStruct((T, B * NS), jnp.float32),
            jax.ShapeDtypeStruct((T, B * NS), jnp.float32),
        ],
        compiler_params=pltpu.CompilerParams(
            dimension_semantics=("parallel",)),
    )(x.astype(jnp.bfloat16), W_in_T.astype(jnp.bfloat16),
      W_cat.astype(jnp.bfloat16), bias)

    n_half = B // B_BLK

    def to_tb(a):
        return (a.reshape(n_half, B_BLK, T, NS)
                 .transpose(0, 2, 1, 3).reshape(n_half, T * B_BLK, NS))

    kt, qt = to_tb(k2), to_tb(q2)
    d_bi = jnp.tile(d_alpha, B)[None, :]
    e8 = jnp.repeat(jnp.eye(B_BLK, dtype=jnp.float32), NS, axis=1)

    grid_b = (n_half, T // TC)
    out, s_final = pl.pallas_call(
        _scan_kernel,
        grid=grid_b,
        in_specs=[
            pl.BlockSpec((1, TC * B_BLK, NS), lambda h, t: (h, t, 0)),
            pl.BlockSpec((1, TC * B_BLK, NS), lambda h, t: (h, t, 0)),
            pl.BlockSpec((TC, LW), lambda h, t: (t, h)),
            pl.BlockSpec((TC, LW), lambda h, t: (t, h)),
            pl.BlockSpec((1, LW), lambda h, t: (0, h)),
            pl.BlockSpec((B_BLK, LW), lambda h, t: (0, 0)),
            pl.BlockSpec((NS, DIM), lambda h, t: (0, 0)),
        ],
        out_specs=[
            pl.BlockSpec((B_BLK, TC, DIM), lambda h, t: (h, t, 0)),
            pl.BlockSpec((B_BLK, NS, NS), lambda h, t: (h, 0, 0)),
        ],
        out_shape=[
            jax.ShapeDtypeStruct((B, T, DIM), jnp.float32),
            jax.ShapeDtypeStruct((B, NS, NS), jnp.float32),
        ],
        scratch_shapes=[
            pltpu.VMEM((NS, LW), jnp.float32),
            pltpu.VMEM((TC, LW), jnp.float32),
        ],
        compiler_params=pltpu.CompilerParams(
            dimension_semantics=("parallel", "arbitrary")),
    )(kt, qt, v_rows, ax_rows, d_bi, e8, W_out.T)

    return out, s_final


# speedup vs baseline: 1.1360x; 1.1360x over previous
import jax
import jax.numpy as jnp
from jax.experimental import pallas as pl
from jax.experimental.pallas import tpu as pltpu

DIM = 1024
NS = 64
PROJ_ROWS = 512
B_BLK = 8
LW = B_BLK * NS
TC = 256


def _proj_kernel(x_ref, w_in_ref, w_cat_ref, bias_ref,
                 k_ref, q_ref, v_ref, ax_ref):
    xb = x_ref[...].reshape(2 * PROJ_ROWS, DIM)
    xp = jnp.dot(xb, w_in_ref[...], preferred_element_type=jnp.float32)
    xp = xp * jax.nn.sigmoid(xp)
    kvqa = jnp.dot(xp, w_cat_ref[...], preferred_element_type=jnp.float32)
    kvqa = kvqa + bias_ref[...]
    k_ref[...] = kvqa[:, 0:NS].reshape(2, PROJ_ROWS, NS)
    q_ref[...] = kvqa[:, 2 * NS:3 * NS].reshape(2, PROJ_ROWS, NS)
    v_ref[...] = jnp.concatenate(
        [kvqa[0:PROJ_ROWS, NS:2 * NS],
         kvqa[PROJ_ROWS:2 * PROJ_ROWS, NS:2 * NS]], axis=1)
    ax_ref[...] = jnp.concatenate(
        [kvqa[0:PROJ_ROWS, 3 * NS:4 * NS],
         kvqa[PROJ_ROWS:2 * PROJ_ROWS, 3 * NS:4 * NS]], axis=1)


def _scan_kernel(kt_ref, qt_ref, v_ref, ax_ref, d_ref, e8_ref, wout_ref,
                 out_ref, sfin_ref, s_ref, o_ref):
    @pl.when(pl.program_id(1) == 0)
    def _():
        s_ref[...] = jnp.zeros_like(s_ref)

    d_row = d_ref[...]
    e8 = e8_ref[...]
    cdims = (((0,), (0,)), ((), ()))

    def body(tt, S):
        base = tt * B_BLK
        kt_t = kt_ref[0, pl.ds(base, B_BLK), :]
        kbt = jax.lax.dot_general(kt_t, e8, cdims,
                                  preferred_element_type=jnp.float32)
        r = jnp.sum(S * kbt, axis=0, keepdims=True)
        z = ax_ref[pl.ds(tt, 1), :] + d_row * r
        alpha = jax.nn.sigmoid(z)
        w = (1.0 - alpha) * v_ref[pl.ds(tt, 1), :]
        S = alpha * S + w * kbt
        qt_t = qt_ref[0, pl.ds(base, B_BLK), :]
        qbt = jax.lax.dot_general(qt_t, e8, cdims,
                                  preferred_element_type=jnp.float32)
        o = jnp.sum(S * qbt, axis=0, keepdims=True)
        o = o * o * jax.nn.sigmoid(o)
        o_ref[pl.ds(tt, 1), :] = o
        return S

    S = jax.lax.fori_loop(0, TC, body, s_ref[...], unroll=128)
    s_ref[...] = S

    @pl.when(pl.program_id(1) == pl.num_programs(1) - 1)
    def _():
        for b in range(B_BLK):
            sfin_ref[b, :, :] = S[:, b * NS:(b + 1) * NS].T

    cell = jnp.concatenate(
        [o_ref[:, b * NS:(b + 1) * NS] for b in range(B_BLK)], axis=0)
    out = jnp.dot(cell, wout_ref[...], preferred_element_type=jnp.float32)
    out_ref[...] = out.reshape(B_BLK, TC, DIM)


def kernel(x, W_in, W_k, W_v, W_q, W_alpha, d_alpha, b_alpha, W_out):
    B, T, D = x.shape
    W_in_T = W_in.T
    W_cat = jnp.concatenate([W_k, W_v, W_q, W_alpha], axis=0).T
    bias = jnp.concatenate(
        [jnp.zeros((3 * NS,), jnp.float32), b_alpha])[None, :]

    tpb = T // PROJ_ROWS
    grid_a = ((B // 2) * tpb,)
    nr_spec = pl.BlockSpec((2, PROJ_ROWS, NS),
                           lambda i: (i // tpb, i % tpb, 0))
    rows_spec = pl.BlockSpec((PROJ_ROWS, 2 * NS), lambda i: (i % tpb, i // tpb))
    k2, q2, v_rows, ax_rows = pl.pallas_call(
        _proj_kernel,
        grid=grid_a,
        in_specs=[
            pl.BlockSpec((2, PROJ_ROWS, DIM), lambda i: (i // tpb, i % tpb, 0)),
            pl.BlockSpec((DIM, DIM), lambda i: (0, 0)),
            pl.BlockSpec((DIM, 4 * NS), lambda i: (0, 0)),
            pl.BlockSpec((1, 4 * NS), lambda i: (0, 0)),
        ],
        out_specs=[nr_spec, nr_spec, rows_spec, rows_spec],
        out_shape=[
            jax.ShapeDtypeStruct((B, T, NS), jnp.float32),
            jax.ShapeDtypeStruct((B, T, NS), jnp.float32),
            jax.ShapeDtypeStruct((T, B * NS), jnp.float32),
            jax.ShapeDtypeStruct((T, B * NS), jnp.float32),
        ],
        compiler_params=pltpu.CompilerParams(
            dimension_semantics=("parallel",)),
    )(x, W_in_T, W_cat, bias)

    n_half = B // B_BLK

    def to_tb(a):
        return (a.reshape(n_half, B_BLK, T, NS)
                 .transpose(0, 2, 1, 3).reshape(n_half, T * B_BLK, NS))

    kt, qt = to_tb(k2), to_tb(q2)
    d_bi = jnp.tile(d_alpha, B)[None, :]
    e8 = jnp.repeat(jnp.eye(B_BLK, dtype=jnp.float32), NS, axis=1)

    grid_b = (n_half, T // TC)
    out, s_final = pl.pallas_call(
        _scan_kernel,
        grid=grid_b,
        in_specs=[
            pl.BlockSpec((1, TC * B_BLK, NS), lambda h, t: (h, t, 0)),
            pl.BlockSpec((1, TC * B_BLK, NS), lambda h, t: (h, t, 0)),
            pl.BlockSpec((TC, LW), lambda h, t: (t, h)),
            pl.BlockSpec((TC, LW), lambda h, t: (t, h)),
            pl.BlockSpec((1, LW), lambda h, t: (0, h)),
            pl.BlockSpec((B_BLK, LW), lambda h, t: (0, 0)),
            pl.BlockSpec((NS, DIM), lambda h, t: (0, 0)),
        ],
        out_specs=[
            pl.BlockSpec((B_BLK, TC, DIM), lambda h, t: (h, t, 0)),
            pl.BlockSpec((B_BLK, NS, NS), lambda h, t: (h, 0, 0)),
        ],
        out_shape=[
            jax.ShapeDtypeStruct((B, T, DIM), jnp.float32),
            jax.ShapeDtypeStruct((B, NS, NS), jnp.float32),
        ],
        scratch_shapes=[
            pltpu.VMEM((NS, LW), jnp.float32),
            pltpu.VMEM((TC, LW), jnp.float32),
        ],
        compiler_params=pltpu.CompilerParams(
            dimension_semantics=("parallel", "arbitrary")),
    )(kt, qt, v_rows, ax_rows, d_bi, e8, W_out.T)

    return out, s_final


# speedup vs baseline: 1.1413x; 1.0047x over previous
import jax
import jax.numpy as jnp
from jax.experimental import pallas as pl
from jax.experimental.pallas import tpu as pltpu

DIM = 1024
NS = 64
PROJ_ROWS = 1024
B_BLK = 8
LW = B_BLK * NS
TC = 256


def _proj_kernel(x_ref, w_in_ref, w_cat_ref, bias_ref,
                 k_ref, q_ref, v_ref, ax_ref):
    xb = x_ref[...].reshape(2 * PROJ_ROWS, DIM)
    xp = jnp.dot(xb, w_in_ref[...], preferred_element_type=jnp.float32)
    xp = xp * jax.nn.sigmoid(xp)
    kvqa = jnp.dot(xp, w_cat_ref[...], preferred_element_type=jnp.float32)
    kvqa = kvqa + bias_ref[...]
    k_ref[...] = kvqa[:, 0:NS].reshape(2, PROJ_ROWS, NS)
    q_ref[...] = kvqa[:, 2 * NS:3 * NS].reshape(2, PROJ_ROWS, NS)
    v_ref[...] = jnp.concatenate(
        [kvqa[0:PROJ_ROWS, NS:2 * NS],
         kvqa[PROJ_ROWS:2 * PROJ_ROWS, NS:2 * NS]], axis=1)
    ax_ref[...] = jnp.concatenate(
        [kvqa[0:PROJ_ROWS, 3 * NS:4 * NS],
         kvqa[PROJ_ROWS:2 * PROJ_ROWS, 3 * NS:4 * NS]], axis=1)


def _scan_kernel(kt_ref, qt_ref, v_ref, ax_ref, d_ref, e8_ref, wout_ref,
                 out_ref, sfin_ref, s_ref, o_ref):
    @pl.when(pl.program_id(1) == 0)
    def _():
        s_ref[...] = jnp.zeros_like(s_ref)

    d_row = d_ref[...]
    e8 = e8_ref[...]
    cdims = (((0,), (0,)), ((), ()))

    def body(tt, S):
        base = tt * B_BLK
        kt_t = kt_ref[0, pl.ds(base, B_BLK), :]
        kbt = jax.lax.dot_general(kt_t, e8, cdims,
                                  preferred_element_type=jnp.float32)
        r = jnp.sum(S * kbt, axis=0, keepdims=True)
        z = ax_ref[pl.ds(tt, 1), :] + d_row * r
        alpha = jax.nn.sigmoid(z)
        w = (1.0 - alpha) * v_ref[pl.ds(tt, 1), :]
        S = alpha * S + w * kbt
        qt_t = qt_ref[0, pl.ds(base, B_BLK), :]
        qbt = jax.lax.dot_general(qt_t, e8, cdims,
                                  preferred_element_type=jnp.float32)
        o = jnp.sum(S * qbt, axis=0, keepdims=True)
        o = o * o * jax.nn.sigmoid(o)
        o_ref[pl.ds(tt, 1), :] = o
        return S

    S = jax.lax.fori_loop(0, TC, body, s_ref[...], unroll=128)
    s_ref[...] = S

    @pl.when(pl.program_id(1) == pl.num_programs(1) - 1)
    def _():
        for b in range(B_BLK):
            sfin_ref[b, :, :] = S[:, b * NS:(b + 1) * NS].T

    cell = jnp.concatenate(
        [o_ref[:, b * NS:(b + 1) * NS] for b in range(B_BLK)], axis=0)
    out = jnp.dot(cell, wout_ref[...], preferred_element_type=jnp.float32)
    out_ref[...] = out.reshape(B_BLK, TC, DIM)


def kernel(x, W_in, W_k, W_v, W_q, W_alpha, d_alpha, b_alpha, W_out):
    B, T, D = x.shape
    W_in_T = W_in.T
    W_cat = jnp.concatenate([W_k, W_v, W_q, W_alpha], axis=0).T
    bias = jnp.concatenate(
        [jnp.zeros((3 * NS,), jnp.float32), b_alpha])[None, :]

    tpb = T // PROJ_ROWS
    grid_a = ((B // 2) * tpb,)
    nr_spec = pl.BlockSpec((2, PROJ_ROWS, NS),
                           lambda i: (i // tpb, i % tpb, 0))
    rows_spec = pl.BlockSpec((PROJ_ROWS, 2 * NS), lambda i: (i % tpb, i // tpb))
    k2, q2, v_rows, ax_rows = pl.pallas_call(
        _proj_kernel,
        grid=grid_a,
        in_specs=[
            pl.BlockSpec((2, PROJ_ROWS, DIM), lambda i: (i // tpb, i % tpb, 0)),
            pl.BlockSpec((DIM, DIM), lambda i: (0, 0)),
            pl.BlockSpec((DIM, 4 * NS), lambda i: (0, 0)),
            pl.BlockSpec((1, 4 * NS), lambda i: (0, 0)),
        ],
        out_specs=[nr_spec, nr_spec, rows_spec, rows_spec],
        out_shape=[
            jax.ShapeDtypeStruct((B, T, NS), jnp.float32),
            jax.ShapeDtypeStruct((B, T, NS), jnp.float32),
            jax.ShapeDtypeStruct((T, B * NS), jnp.float32),
            jax.ShapeDtypeStruct((T, B * NS), jnp.float32),
        ],
        compiler_params=pltpu.CompilerParams(
            dimension_semantics=("parallel",)),
    )(x, W_in_T, W_cat, bias)

    n_half = B // B_BLK

    def to_tb(a):
        return (a.reshape(n_half, B_BLK, T, NS)
                 .transpose(0, 2, 1, 3).reshape(n_half, T * B_BLK, NS))

    kt, qt = to_tb(k2), to_tb(q2)
    d_bi = jnp.tile(d_alpha, B)[None, :]
    e8 = jnp.repeat(jnp.eye(B_BLK, dtype=jnp.float32), NS, axis=1)

    grid_b = (n_half, T // TC)
    out, s_final = pl.pallas_call(
        _scan_kernel,
        grid=grid_b,
        in_specs=[
            pl.BlockSpec((1, TC * B_BLK, NS), lambda h, t: (h, t, 0)),
            pl.BlockSpec((1, TC * B_BLK, NS), lambda h, t: (h, t, 0)),
            pl.BlockSpec((TC, LW), lambda h, t: (t, h)),
            pl.BlockSpec((TC, LW), lambda h, t: (t, h)),
            pl.BlockSpec((1, LW), lambda h, t: (0, h)),
            pl.BlockSpec((B_BLK, LW), lambda h, t: (0, 0)),
            pl.BlockSpec((NS, DIM), lambda h, t: (0, 0)),
        ],
        out_specs=[
            pl.BlockSpec((B_BLK, TC, DIM), lambda h, t: (h, t, 0)),
            pl.BlockSpec((B_BLK, NS, NS), lambda h, t: (h, 0, 0)),
        ],
        out_shape=[
            jax.ShapeDtypeStruct((B, T, DIM), jnp.float32),
            jax.ShapeDtypeStruct((B, NS, NS), jnp.float32),
        ],
        scratch_shapes=[
            pltpu.VMEM((NS, LW), jnp.float32),
            pltpu.VMEM((TC, LW), jnp.float32),
        ],
        compiler_params=pltpu.CompilerParams(
            dimension_semantics=("parallel", "arbitrary")),
    )(kt, qt, v_rows, ax_rows, d_bi, e8, W_out.T)

    return out, s_final
